# Initial kernel scaffold; baseline (speedup 1.0000x reference)
import jax
import jax.numpy as jnp
from jax.experimental import pallas as pl


def kernel(x, edge_index, params):
    raise NotImplementedError("write your pallas kernel here")


# speedup vs baseline: 4.4852x; 4.4852x over previous
import functools

import jax
import jax.numpy as jnp
from jax import lax
from jax.experimental import pallas as pl
from jax.experimental.pallas import tpu as pltpu
from jax.experimental.pallas import tpu_sc as plsc

NC = 2
NS = 16
CH = 80
W = 128


@functools.cache
def _make_aggregate(n, e):
    nw = NC * NS
    epw = e // nw
    assert e % nw == 0 and epw % CH == 0
    nch = epw // CH
    npad = -(-n // (8 * NS)) * (8 * NS)
    rps = npad // NS

    mesh = plsc.VectorSubcoreMesh(core_axis_name="c", subcore_axis_name="s")

    def body(x_hbm, src_hbm, dst_hbm, zero_hbm, out_hbm,
             acc_sh, src_v, dst_v, rows_v):
        c = lax.axis_index("c")
        s = lax.axis_index("s")
        wid = s * NC + c

        pltpu.sync_copy(zero_hbm.at[pl.ds(s * rps, rps)],
                        acc_sh.at[pl.ds(s * rps, rps)])
        plsc.subcore_barrier()

        base0 = wid * epw

        def chunk(i, carry):
            base = pl.multiple_of(base0 + i * CH, 8)
            pltpu.sync_copy(src_hbm.at[pl.ds(base, CH)], src_v)
            pltpu.sync_copy(dst_hbm.at[pl.ds(base, CH)], dst_v)
            pltpu.sync_copy(x_hbm.at[src_v], rows_v)
            pltpu.sync_copy(rows_v, acc_sh.at[dst_v], add=True)
            return carry

        lax.fori_loop(0, nch, chunk, 0)
        plsc.subcore_barrier()

        pltpu.sync_copy(acc_sh.at[pl.ds(s * rps, rps)],
                        out_hbm.at[c, pl.ds(s * rps, rps)])

    return pl.kernel(
        body,
        out_type=jax.ShapeDtypeStruct((NC, npad, W), jnp.float32),
        mesh=mesh,
        scratch_types=[
            pltpu.VMEM_SHARED((npad, W), jnp.float32),
            pltpu.VMEM((CH,), jnp.int32),
            pltpu.VMEM((CH,), jnp.int32),
            pltpu.VMEM((CH, W), jnp.float32),
        ],
    )


def _bn_in(t, g, b):
    mu = jnp.mean(t, axis=0, keepdims=True)
    var = jnp.mean((t - mu) * (t - mu), axis=0, keepdims=True)
    return (t - mu) / jnp.sqrt(var + 1e-5) * g + b


@functools.cache
def _make_layer(n):
    def body(h_ref, a0_ref, a1_ref, eps_ref,
             w1_ref, b1_ref, gm_ref, bm_ref,
             w2_ref, b2_ref, g2_ref, bb2_ref, out_ref):
        z = h_ref[...] * eps_ref[0, 0] + a0_ref[:n] + a1_ref[:n]
        t = jnp.dot(z, w1_ref[...], preferred_element_type=jnp.float32)
        t = _bn_in(t + b1_ref[...], gm_ref[...], bm_ref[...])
        t = jnp.maximum(t, 0.0)
        u = jnp.dot(t, w2_ref[...], preferred_element_type=jnp.float32)
        u = _bn_in(u + b2_ref[...], g2_ref[...], bb2_ref[...])
        out_ref[...] = jnp.maximum(u, 0.0)

    vmem = pl.BlockSpec(memory_space=pltpu.VMEM)
    smem = pl.BlockSpec(memory_space=pltpu.SMEM)
    return pl.pallas_call(
        body,
        out_shape=jax.ShapeDtypeStruct((n, W), jnp.float32),
        in_specs=[vmem, vmem, vmem, smem] + [vmem] * 8,
        out_specs=vmem,
    )


@functools.cache
def _make_head(n, do):
    def body(h_ref, w1_ref, b1_ref, g_ref, b_ref, w2_ref, b2_ref, out_ref):
        t = jnp.dot(h_ref[...], w1_ref[...], preferred_element_type=jnp.float32)
        t = _bn_in(t + b1_ref[...], g_ref[...], b_ref[...])
        t = jnp.maximum(t, 0.0)
        u = jnp.dot(t, w2_ref[...], preferred_element_type=jnp.float32)
        u = u + b2_ref[...]
        m = jnp.max(u, axis=-1, keepdims=True)
        ex = jnp.exp(u - m)
        lse = jnp.log(jnp.sum(ex, axis=-1, keepdims=True)) + m
        out_ref[...] = u - lse

    vmem = pl.BlockSpec(memory_space=pltpu.VMEM)
    return pl.pallas_call(
        body,
        out_shape=jax.ShapeDtypeStruct((n, do), jnp.float32),
        in_specs=[vmem] * 7,
        out_specs=vmem,
    )


def _pad_to(a, shape):
    pads = [(0, t - s) for s, t in zip(a.shape, shape)]
    return jnp.pad(a, pads)


def kernel(x, edge_index, params):
    n, d_in = x.shape
    e = edge_index.shape[1]
    assert d_in == W
    src = edge_index[0]
    dst = edge_index[1]
    npad = -(-n // (8 * NS)) * (8 * NS)
    zero = jnp.zeros((npad, W), jnp.float32)

    aggregate = _make_aggregate(n, e)
    layer = _make_layer(n)

    h = x
    for p in params["convs"]:
        acc = aggregate(h, src, dst, zero)
        dm = p["W1"].shape[1]
        eps = jnp.reshape(1.0 + p["eps"], (1, 1))
        h = layer(
            h, acc[0], acc[1], eps,
            _pad_to(p["W1"], (W, W)), _pad_to(jnp.reshape(p["b1"], (1, dm)), (1, W)),
            _pad_to(jnp.reshape(p["bn_mid_g"], (1, dm)), (1, W)),
            _pad_to(jnp.reshape(p["bn_mid_b"], (1, dm)), (1, W)),
            _pad_to(p["W2"], (W, W)),
            _pad_to(jnp.reshape(p["b2"], (1, p["W2"].shape[1])), (1, W)),
            _pad_to(jnp.reshape(p["bn_g"], (1, p["W2"].shape[1])), (1, W)),
            _pad_to(jnp.reshape(p["bn_b"], (1, p["W2"].shape[1])), (1, W)),
        )

    d = params["lin1_W"].shape[0]
    do = params["lin2_W"].shape[1]
    return _make_head(n, do)(
        h, _pad_to(params["lin1_W"], (W, d)),
        jnp.reshape(params["lin1_b"], (1, d)),
        jnp.reshape(params["bn1_g"], (1, d)), jnp.reshape(params["bn1_b"], (1, d)),
        params["lin2_W"], jnp.reshape(params["lin2_b"], (1, do)),
    )


# speedup vs baseline: 10.9221x; 2.4352x over previous
import functools

import jax
import jax.numpy as jnp
from jax import lax
from jax.experimental import pallas as pl
from jax.experimental.pallas import tpu as pltpu
from jax.experimental.pallas import tpu_sc as plsc

NC = 2
NS = 16
CH = 128
GRP = 8
W = 128


@functools.cache
def _make_aggregate(n, e):
    nw = NC * NS
    epw = e // nw
    nch = epw // CH
    ng = nch // GRP
    assert e % (nw * GRP * CH) == 0 and ng >= 3
    npad = -(-n // (8 * NS)) * (8 * NS)
    rps = npad // NS

    mesh = plsc.VectorSubcoreMesh(core_axis_name="c", subcore_axis_name="s")

    def body(x_hbm, src_hbm, dst_hbm, zero_hbm, out_hbm,
             acc_sh, src_r, dst_r, b0, b1,
             gs0, gs1, ss0, ss1, ise, ide):
        c = lax.axis_index("c")
        s = lax.axis_index("s")
        wid = s * NC + c
        bufs = [b0, b1]
        gs = [gs0, gs1]
        ss = [ss0, ss1]

        def i_start(g, r):
            off = pl.multiple_of(g * GRP, GRP)
            pltpu.async_copy(src_hbm.at[wid, pl.ds(off, GRP)], src_r.at[r], ise)
            pltpu.async_copy(dst_hbm.at[wid, pl.ds(off, GRP)], dst_r.at[r], ide)

        def i_wait(g, r):
            off = pl.multiple_of(g * GRP, GRP)
            pltpu.make_async_copy(src_hbm.at[wid, pl.ds(off, GRP)],
                                  src_r.at[r], ise).wait()
            pltpu.make_async_copy(dst_hbm.at[wid, pl.ds(off, GRP)],
                                  dst_r.at[r], ide).wait()

        def g_start(gr, j, b):
            pltpu.async_copy(x_hbm.at[src_r.at[gr, j]], bufs[b], gs[b])

        def slot(gr, j, b, swait=None, gst=None):
            bn = (b + 1) % 2
            if swait is not None:
                pltpu.make_async_copy(bufs[bn], acc_sh.at[dst_r.at[swait]],
                                      ss[bn]).wait()
            if gst is not None:
                g_start(gst[0], gst[1], bn)
            pltpu.make_async_copy(x_hbm.at[src_r.at[gr, j]], bufs[b],
                                  gs[b]).wait()
            pltpu.async_copy(bufs[b], acc_sh.at[dst_r.at[gr, j]], ss[b],
                             add=True)

        def group(g, gr, go, first=False, last=False):
            slot(gr, 0, 0,
                 swait=None if first else (go, GRP - 1),
                 gst=(gr, 1))
            if not last:
                i_start(g + 1, go)
            for j in range(1, GRP - 1):
                slot(gr, j, j % 2, swait=(gr, j - 1), gst=(gr, j + 1))
            if not last:
                i_wait(g + 1, go)
            slot(gr, GRP - 1, (GRP - 1) % 2, swait=(gr, GRP - 2),
                 gst=None if last else (go, 0))

        pltpu.sync_copy(zero_hbm.at[pl.ds(s * rps, rps)],
                        acc_sh.at[pl.ds(s * rps, rps)])
        i_start(0, 0)
        i_wait(0, 0)
        plsc.subcore_barrier()

        g_start(0, 0, 0)
        group(0, 0, 1, first=True)

        def mid(g, carry):
            gr = lax.rem(g, 2)
            group(g, gr, 1 - gr)
            return carry

        lax.fori_loop(1, ng - 1, mid, 0)

        group(ng - 1, (ng - 1) % 2, 1 - (ng - 1) % 2, last=True)
        pltpu.make_async_copy(
            bufs[(GRP - 1) % 2],
            acc_sh.at[dst_r.at[(ng - 1) % 2, GRP - 1]],
            ss[(GRP - 1) % 2]).wait()

        plsc.subcore_barrier()
        pltpu.sync_copy(acc_sh.at[pl.ds(s * rps, rps)],
                        out_hbm.at[c, pl.ds(s * rps, rps)])

    return pl.kernel(
        body,
        out_type=jax.ShapeDtypeStruct((NC, npad, W), jnp.float32),
        mesh=mesh,
        scratch_types=[
            pltpu.VMEM_SHARED((npad, W), jnp.float32),
            pltpu.VMEM((2, GRP, CH), jnp.int32),
            pltpu.VMEM((2, GRP, CH), jnp.int32),
            pltpu.VMEM((CH, W), jnp.float32),
            pltpu.VMEM((CH, W), jnp.float32),
        ] + [pltpu.SemaphoreType.DMA] * 6,
    )


def _bn_in(t, g, b):
    mu = jnp.mean(t, axis=0, keepdims=True)
    var = jnp.mean((t - mu) * (t - mu), axis=0, keepdims=True)
    return (t - mu) / jnp.sqrt(var + 1e-5) * g + b


@functools.cache
def _make_layer(n):
    def body(h_ref, a0_ref, a1_ref, eps_ref,
             w1_ref, b1_ref, gm_ref, bm_ref,
             w2_ref, b2_ref, g2_ref, bb2_ref, out_ref):
        z = h_ref[...] * eps_ref[0, 0] + a0_ref[:n] + a1_ref[:n]
        t = jnp.dot(z, w1_ref[...], preferred_element_type=jnp.float32)
        t = _bn_in(t + b1_ref[...], gm_ref[...], bm_ref[...])
        t = jnp.maximum(t, 0.0)
        u = jnp.dot(t, w2_ref[...], preferred_element_type=jnp.float32)
        u = _bn_in(u + b2_ref[...], g2_ref[...], bb2_ref[...])
        out_ref[...] = jnp.maximum(u, 0.0)

    vmem = pl.BlockSpec(memory_space=pltpu.VMEM)
    smem = pl.BlockSpec(memory_space=pltpu.SMEM)
    return pl.pallas_call(
        body,
        out_shape=jax.ShapeDtypeStruct((n, W), jnp.float32),
        in_specs=[vmem, vmem, vmem, smem] + [vmem] * 8,
        out_specs=vmem,
    )


@functools.cache
def _make_head(n, do):
    def body(h_ref, w1_ref, b1_ref, g_ref, b_ref, w2_ref, b2_ref, out_ref):
        t = jnp.dot(h_ref[...], w1_ref[...], preferred_element_type=jnp.float32)
        t = _bn_in(t + b1_ref[...], g_ref[...], b_ref[...])
        t = jnp.maximum(t, 0.0)
        u = jnp.dot(t, w2_ref[...], preferred_element_type=jnp.float32)
        u = u + b2_ref[...]
        m = jnp.max(u, axis=-1, keepdims=True)
        ex = jnp.exp(u - m)
        lse = jnp.log(jnp.sum(ex, axis=-1, keepdims=True)) + m
        out_ref[...] = u - lse

    vmem = pl.BlockSpec(memory_space=pltpu.VMEM)
    return pl.pallas_call(
        body,
        out_shape=jax.ShapeDtypeStruct((n, do), jnp.float32),
        in_specs=[vmem] * 7,
        out_specs=vmem,
    )


def _pad_to(a, shape):
    pads = [(0, t - s) for s, t in zip(a.shape, shape)]
    return jnp.pad(a, pads)


def kernel(x, edge_index, params):
    n, d_in = x.shape
    e = edge_index.shape[1]
    assert d_in == W
    npad = -(-n // (8 * NS)) * (8 * NS)
    zero = jnp.zeros((npad, W), jnp.float32)

    nw = NC * NS
    quant = nw * GRP * CH
    ep = -(-e // quant) * quant
    pad = ep - e
    pad_idx = jnp.arange(pad, dtype=jnp.int32)
    src = jnp.concatenate([edge_index[0], pad_idx % n]).reshape(nw, ep // (nw * CH), CH)
    dst = jnp.concatenate([edge_index[1], n + pad_idx % (npad - n)]
                          ).reshape(nw, ep // (nw * CH), CH)

    aggregate = _make_aggregate(n, ep)
    layer = _make_layer(n)

    h = x
    for p in params["convs"]:
        acc = aggregate(h, src, dst, zero)
        dm = p["W1"].shape[1]
        eps = jnp.reshape(1.0 + p["eps"], (1, 1))
        h = layer(
            h, acc[0], acc[1], eps,
            _pad_to(p["W1"], (W, W)), _pad_to(jnp.reshape(p["b1"], (1, dm)), (1, W)),
            _pad_to(jnp.reshape(p["bn_mid_g"], (1, dm)), (1, W)),
            _pad_to(jnp.reshape(p["bn_mid_b"], (1, dm)), (1, W)),
            _pad_to(p["W2"], (W, W)),
            _pad_to(jnp.reshape(p["b2"], (1, p["W2"].shape[1])), (1, W)),
            _pad_to(jnp.reshape(p["bn_g"], (1, p["W2"].shape[1])), (1, W)),
            _pad_to(jnp.reshape(p["bn_b"], (1, p["W2"].shape[1])), (1, W)),
        )

    d = params["lin1_W"].shape[0]
    do = params["lin2_W"].shape[1]
    return _make_head(n, do)(
        h, _pad_to(params["lin1_W"], (W, d)),
        jnp.reshape(params["lin1_b"], (1, d)),
        jnp.reshape(params["bn1_g"], (1, d)), jnp.reshape(params["bn1_b"], (1, d)),
        params["lin2_W"], jnp.reshape(params["lin2_b"], (1, do)),
    )


# speedup vs baseline: 11.0532x; 1.0120x over previous
import functools

import jax
import jax.numpy as jnp
from jax import lax
from jax.experimental import pallas as pl
from jax.experimental.pallas import tpu as pltpu
from jax.experimental.pallas import tpu_sc as plsc

NC = 2
NS = 16
CH = 64
GRP = 16
NB = 4
W = 128


@functools.cache
def _make_aggregate(n, e):
    nw = NC * NS
    epw = e // nw
    nch = epw // CH
    ng = nch // GRP
    assert e % (nw * GRP * CH) == 0 and ng >= 3
    npad = -(-n // (8 * NS)) * (8 * NS)
    rps = npad // NS

    mesh = plsc.VectorSubcoreMesh(core_axis_name="c", subcore_axis_name="s")

    def body(x_hbm, src_hbm, dst_hbm, zero_hbm, out_hbm,
             acc_sh, src_r, dst_r, b0, b1, b2, b3,
             gs0, gs1, gs2, gs3, ss0, ss1, ss2, ss3, ise, ide):
        c = lax.axis_index("c")
        s = lax.axis_index("s")
        wid = s * NC + c
        bufs = [b0, b1, b2, b3]
        gs = [gs0, gs1, gs2, gs3]
        ss = [ss0, ss1, ss2, ss3]

        def i_start(g, r):
            off = pl.multiple_of(g * GRP, GRP)
            pltpu.async_copy(src_hbm.at[wid, pl.ds(off, GRP)], src_r.at[r], ise)
            pltpu.async_copy(dst_hbm.at[wid, pl.ds(off, GRP)], dst_r.at[r], ide)

        def i_wait(g, r):
            off = pl.multiple_of(g * GRP, GRP)
            pltpu.make_async_copy(src_hbm.at[wid, pl.ds(off, GRP)],
                                  src_r.at[r], ise).wait()
            pltpu.make_async_copy(dst_hbm.at[wid, pl.ds(off, GRP)],
                                  dst_r.at[r], ide).wait()

        def g_start(gr, j, b):
            pltpu.async_copy(x_hbm.at[src_r.at[gr, j]], bufs[b], gs[b])

        def slot(gr, j, b, swait=None, gst=None):
            bn = (b + 2) % NB
            if swait is not None:
                pltpu.make_async_copy(bufs[bn], acc_sh.at[dst_r.at[swait]],
                                      ss[bn]).wait()
            if gst is not None:
                g_start(gst[0], gst[1], bn)
            pltpu.make_async_copy(x_hbm.at[src_r.at[gr, j]], bufs[b],
                                  gs[b]).wait()
            pltpu.async_copy(bufs[b], acc_sh.at[dst_r.at[gr, j]], ss[b],
                             add=True)

        def group(g, gr, go, first=False, last=False):
            slot(gr, 0, 0,
                 swait=None if first else (go, GRP - 2), gst=(gr, 2))
            slot(gr, 1, 1,
                 swait=None if first else (go, GRP - 1), gst=(gr, 3))
            if not last:
                i_start(g + 1, go)
            for j in range(2, GRP - 2):
                slot(gr, j, j % NB, swait=(gr, j - 2), gst=(gr, j + 2))
            if not last:
                i_wait(g + 1, go)
            slot(gr, GRP - 2, (GRP - 2) % NB, swait=(gr, GRP - 4),
                 gst=None if last else (go, 0))
            slot(gr, GRP - 1, (GRP - 1) % NB, swait=(gr, GRP - 3),
                 gst=None if last else (go, 1))

        pltpu.sync_copy(zero_hbm.at[pl.ds(s * rps, rps)],
                        acc_sh.at[pl.ds(s * rps, rps)])
        i_start(0, 0)
        i_wait(0, 0)
        plsc.subcore_barrier()

        g_start(0, 0, 0)
        g_start(0, 1, 1)
        group(0, 0, 1, first=True)

        def mid(g, carry):
            gr = lax.rem(g, 2)
            group(g, gr, 1 - gr)
            return carry

        lax.fori_loop(1, ng - 1, mid, 0)

        gl = (ng - 1) % 2
        group(ng - 1, gl, 1 - gl, last=True)
        pltpu.make_async_copy(
            bufs[(GRP - 2) % NB],
            acc_sh.at[dst_r.at[gl, GRP - 2]],
            ss[(GRP - 2) % NB]).wait()
        pltpu.make_async_copy(
            bufs[(GRP - 1) % NB],
            acc_sh.at[dst_r.at[gl, GRP - 1]],
            ss[(GRP - 1) % NB]).wait()

        plsc.subcore_barrier()
        pltpu.sync_copy(acc_sh.at[pl.ds(s * rps, rps)],
                        out_hbm.at[c, pl.ds(s * rps, rps)])

    return pl.kernel(
        body,
        out_type=jax.ShapeDtypeStruct((NC, npad, W), jnp.float32),
        mesh=mesh,
        scratch_types=[
            pltpu.VMEM_SHARED((npad, W), jnp.float32),
            pltpu.VMEM((2, GRP, CH), jnp.int32),
            pltpu.VMEM((2, GRP, CH), jnp.int32),
        ] + [pltpu.VMEM((CH, W), jnp.float32)] * NB
          + [pltpu.SemaphoreType.DMA] * (2 * NB + 2),
    )


def _bn_in(t, g, b):
    mu = jnp.mean(t, axis=0, keepdims=True)
    var = jnp.mean((t - mu) * (t - mu), axis=0, keepdims=True)
    return (t - mu) / jnp.sqrt(var + 1e-5) * g + b


@functools.cache
def _make_layer(n):
    def body(h_ref, a0_ref, a1_ref, eps_ref,
             w1_ref, b1_ref, gm_ref, bm_ref,
             w2_ref, b2_ref, g2_ref, bb2_ref, out_ref):
        z = h_ref[...] * eps_ref[0, 0] + a0_ref[:n] + a1_ref[:n]
        t = jnp.dot(z, w1_ref[...], preferred_element_type=jnp.float32)
        t = _bn_in(t + b1_ref[...], gm_ref[...], bm_ref[...])
        t = jnp.maximum(t, 0.0)
        u = jnp.dot(t, w2_ref[...], preferred_element_type=jnp.float32)
        u = _bn_in(u + b2_ref[...], g2_ref[...], bb2_ref[...])
        out_ref[...] = jnp.maximum(u, 0.0)

    vmem = pl.BlockSpec(memory_space=pltpu.VMEM)
    smem = pl.BlockSpec(memory_space=pltpu.SMEM)
    return pl.pallas_call(
        body,
        out_shape=jax.ShapeDtypeStruct((n, W), jnp.float32),
        in_specs=[vmem, vmem, vmem, smem] + [vmem] * 8,
        out_specs=vmem,
    )


@functools.cache
def _make_head(n, do):
    def body(h_ref, w1_ref, b1_ref, g_ref, b_ref, w2_ref, b2_ref, out_ref):
        t = jnp.dot(h_ref[...], w1_ref[...], preferred_element_type=jnp.float32)
        t = _bn_in(t + b1_ref[...], g_ref[...], b_ref[...])
        t = jnp.maximum(t, 0.0)
        u = jnp.dot(t, w2_ref[...], preferred_element_type=jnp.float32)
        u = u + b2_ref[...]
        m = jnp.max(u, axis=-1, keepdims=True)
        ex = jnp.exp(u - m)
        lse = jnp.log(jnp.sum(ex, axis=-1, keepdims=True)) + m
        out_ref[...] = u - lse

    vmem = pl.BlockSpec(memory_space=pltpu.VMEM)
    return pl.pallas_call(
        body,
        out_shape=jax.ShapeDtypeStruct((n, do), jnp.float32),
        in_specs=[vmem] * 7,
        out_specs=vmem,
    )


def _pad_to(a, shape):
    pads = [(0, t - s) for s, t in zip(a.shape, shape)]
    return jnp.pad(a, pads)


def kernel(x, edge_index, params):
    n, d_in = x.shape
    e = edge_index.shape[1]
    assert d_in == W
    npad = -(-n // (8 * NS)) * (8 * NS)
    zero = jnp.zeros((npad, W), jnp.float32)

    nw = NC * NS
    quant = nw * GRP * CH
    ep = -(-e // quant) * quant
    pad = ep - e
    pad_idx = jnp.arange(pad, dtype=jnp.int32)
    src = jnp.concatenate([edge_index[0], pad_idx % n]).reshape(nw, ep // (nw * CH), CH)
    dst = jnp.concatenate([edge_index[1], n + pad_idx % (npad - n)]
                          ).reshape(nw, ep // (nw * CH), CH)

    aggregate = _make_aggregate(n, ep)
    layer = _make_layer(n)

    h = x
    for p in params["convs"]:
        acc = aggregate(h, src, dst, zero)
        dm = p["W1"].shape[1]
        eps = jnp.reshape(1.0 + p["eps"], (1, 1))
        h = layer(
            h, acc[0], acc[1], eps,
            _pad_to(p["W1"], (W, W)), _pad_to(jnp.reshape(p["b1"], (1, dm)), (1, W)),
            _pad_to(jnp.reshape(p["bn_mid_g"], (1, dm)), (1, W)),
            _pad_to(jnp.reshape(p["bn_mid_b"], (1, dm)), (1, W)),
            _pad_to(p["W2"], (W, W)),
            _pad_to(jnp.reshape(p["b2"], (1, p["W2"].shape[1])), (1, W)),
            _pad_to(jnp.reshape(p["bn_g"], (1, p["W2"].shape[1])), (1, W)),
            _pad_to(jnp.reshape(p["bn_b"], (1, p["W2"].shape[1])), (1, W)),
        )

    d = params["lin1_W"].shape[0]
    do = params["lin2_W"].shape[1]
    return _make_head(n, do)(
        h, _pad_to(params["lin1_W"], (W, d)),
        jnp.reshape(params["lin1_b"], (1, d)),
        jnp.reshape(params["bn1_g"], (1, d)), jnp.reshape(params["bn1_b"], (1, d)),
        params["lin2_W"], jnp.reshape(params["lin2_b"], (1, do)),
    )


# speedup vs baseline: 11.7294x; 1.0612x over previous
import functools

import jax
import jax.numpy as jnp
from jax import lax
from jax.experimental import pallas as pl
from jax.experimental.pallas import tpu as pltpu
from jax.experimental.pallas import tpu_sc as plsc

NC = 2
NS = 16
CH = 64
GRP = 16
NB = 4
W = 128


@functools.cache
def _make_aggregate(n, e):
    nw = NC * NS
    epw = e // nw
    nch = epw // CH
    ng = nch // GRP
    assert e % (nw * GRP * CH) == 0 and ng >= 3
    npad = -(-n // (8 * NS)) * (8 * NS)
    rps = npad // NS

    mesh = plsc.VectorSubcoreMesh(core_axis_name="c", subcore_axis_name="s")

    def body(x_hbm, src_hbm, dst_hbm, zero_hbm, out_hbm,
             acc_sh, src_r, dst_r, b0, b1, b2, b3,
             gs0, gs1, gs2, gs3, ss0, ss1, ss2, ss3, ise, ide):
        c = lax.axis_index("c")
        s = lax.axis_index("s")
        wid = s * NC + c
        bufs = [b0, b1, b2, b3]
        gs = [gs0, gs1, gs2, gs3]
        ss = [ss0, ss1, ss2, ss3]

        def i_start(g, r):
            off = pl.multiple_of(g * GRP, GRP)
            pltpu.async_copy(src_hbm.at[wid, pl.ds(off, GRP)], src_r.at[r], ise)
            pltpu.async_copy(dst_hbm.at[wid, pl.ds(off, GRP)], dst_r.at[r], ide)

        def i_wait(g, r):
            off = pl.multiple_of(g * GRP, GRP)
            pltpu.make_async_copy(src_hbm.at[wid, pl.ds(off, GRP)],
                                  src_r.at[r], ise).wait()
            pltpu.make_async_copy(dst_hbm.at[wid, pl.ds(off, GRP)],
                                  dst_r.at[r], ide).wait()

        def g_start(gr, j, b):
            pltpu.async_copy(x_hbm.at[src_r.at[gr, j]], bufs[b], gs[b])

        def slot(gr, j, b, swait=None, gst=None):
            bn = (b + 2) % NB
            if swait is not None:
                pltpu.make_async_copy(bufs[bn], acc_sh.at[dst_r.at[swait]],
                                      ss[bn]).wait()
            if gst is not None:
                g_start(gst[0], gst[1], bn)
            pltpu.make_async_copy(x_hbm.at[src_r.at[gr, j]], bufs[b],
                                  gs[b]).wait()
            pltpu.async_copy(bufs[b], acc_sh.at[dst_r.at[gr, j]], ss[b],
                             add=True)

        def group(g, gr, go, first=False, last=False):
            slot(gr, 0, 0,
                 swait=None if first else (go, GRP - 2), gst=(gr, 2))
            slot(gr, 1, 1,
                 swait=None if first else (go, GRP - 1), gst=(gr, 3))
            if not last:
                i_start(g + 1, go)
            for j in range(2, GRP - 2):
                slot(gr, j, j % NB, swait=(gr, j - 2), gst=(gr, j + 2))
            if not last:
                i_wait(g + 1, go)
            slot(gr, GRP - 2, (GRP - 2) % NB, swait=(gr, GRP - 4),
                 gst=None if last else (go, 0))
            slot(gr, GRP - 1, (GRP - 1) % NB, swait=(gr, GRP - 3),
                 gst=None if last else (go, 1))

        pltpu.sync_copy(zero_hbm.at[pl.ds(s * rps, rps)],
                        acc_sh.at[pl.ds(s * rps, rps)])
        i_start(0, 0)
        i_wait(0, 0)
        plsc.subcore_barrier()

        g_start(0, 0, 0)
        g_start(0, 1, 1)
        group(0, 0, 1, first=True)

        def mid(g, carry):
            gr = lax.rem(g, 2)
            group(g, gr, 1 - gr)
            return carry

        lax.fori_loop(1, ng - 1, mid, 0)

        gl = (ng - 1) % 2
        group(ng - 1, gl, 1 - gl, last=True)
        pltpu.make_async_copy(
            bufs[(GRP - 2) % NB],
            acc_sh.at[dst_r.at[gl, GRP - 2]],
            ss[(GRP - 2) % NB]).wait()
        pltpu.make_async_copy(
            bufs[(GRP - 1) % NB],
            acc_sh.at[dst_r.at[gl, GRP - 1]],
            ss[(GRP - 1) % NB]).wait()

        plsc.subcore_barrier()
        pltpu.sync_copy(acc_sh.at[pl.ds(s * rps, rps)],
                        out_hbm.at[c, pl.ds(s * rps, rps)])

    return pl.kernel(
        body,
        out_type=jax.ShapeDtypeStruct((NC, npad, W), jnp.float32),
        mesh=mesh,
        scratch_types=[
            pltpu.VMEM_SHARED((npad, W), jnp.float32),
            pltpu.VMEM((2, GRP, CH), jnp.int32),
            pltpu.VMEM((2, GRP, CH), jnp.int32),
        ] + [pltpu.VMEM((CH, W), jnp.float32)] * NB
          + [pltpu.SemaphoreType.DMA] * (2 * NB + 2),
    )


def _bn_in(t, g, b):
    mu = jnp.mean(t, axis=0, keepdims=True)
    var = jnp.mean((t - mu) * (t - mu), axis=0, keepdims=True)
    return (t - mu) / jnp.sqrt(var + 1e-5) * g + b


def _layer_math(n, h_ref, a_ref, eps_ref, w1_ref, b1_ref, gm_ref, bm_ref,
                w2_ref, b2_ref, g2_ref, bb2_ref):
    z = h_ref[...] * eps_ref[0, 0] + a_ref[0, :n] + a_ref[1, :n]
    t = jnp.dot(z, w1_ref[...], preferred_element_type=jnp.float32)
    t = _bn_in(t + b1_ref[...], gm_ref[...], bm_ref[...])
    t = jnp.maximum(t, 0.0)
    u = jnp.dot(t, w2_ref[...], preferred_element_type=jnp.float32)
    u = _bn_in(u + b2_ref[...], g2_ref[...], bb2_ref[...])
    return jnp.maximum(u, 0.0)


@functools.cache
def _make_layer(n):
    def body(h_ref, a_ref, eps_ref,
             w1_ref, b1_ref, gm_ref, bm_ref,
             w2_ref, b2_ref, g2_ref, bb2_ref, out_ref):
        out_ref[...] = _layer_math(n, h_ref, a_ref, eps_ref, w1_ref, b1_ref,
                                   gm_ref, bm_ref, w2_ref, b2_ref, g2_ref,
                                   bb2_ref)

    vmem = pl.BlockSpec(memory_space=pltpu.VMEM)
    smem = pl.BlockSpec(memory_space=pltpu.SMEM)
    return pl.pallas_call(
        body,
        out_shape=jax.ShapeDtypeStruct((n, W), jnp.float32),
        in_specs=[vmem, vmem, smem] + [vmem] * 8,
        out_specs=vmem,
    )


@functools.cache
def _make_layer_head(n, do):
    def body(h_ref, a_ref, eps_ref,
             w1_ref, b1_ref, gm_ref, bm_ref,
             w2_ref, b2_ref, g2_ref, bb2_ref,
             hw1_ref, hb1_ref, hg_ref, hb_ref, hw2_ref, hb2_ref, out_ref):
        h = _layer_math(n, h_ref, a_ref, eps_ref, w1_ref, b1_ref, gm_ref,
                        bm_ref, w2_ref, b2_ref, g2_ref, bb2_ref)
        t = jnp.dot(h, hw1_ref[...], preferred_element_type=jnp.float32)
        t = _bn_in(t + hb1_ref[...], hg_ref[...], hb_ref[...])
        t = jnp.maximum(t, 0.0)
        u = jnp.dot(t, hw2_ref[...], preferred_element_type=jnp.float32)
        u = u + hb2_ref[...]
        m = jnp.max(u, axis=-1, keepdims=True)
        ex = jnp.exp(u - m)
        lse = jnp.log(jnp.sum(ex, axis=-1, keepdims=True)) + m
        out_ref[...] = u - lse

    vmem = pl.BlockSpec(memory_space=pltpu.VMEM)
    smem = pl.BlockSpec(memory_space=pltpu.SMEM)
    return pl.pallas_call(
        body,
        out_shape=jax.ShapeDtypeStruct((n, do), jnp.float32),
        in_specs=[vmem, vmem, smem] + [vmem] * 14,
        out_specs=vmem,
    )


def _pad_to(a, shape):
    pads = [(0, t - s) for s, t in zip(a.shape, shape)]
    return jnp.pad(a, pads)


def kernel(x, edge_index, params):
    n, d_in = x.shape
    e = edge_index.shape[1]
    assert d_in == W
    npad = -(-n // (8 * NS)) * (8 * NS)
    zero = jnp.zeros((npad, W), jnp.float32)

    nw = NC * NS
    quant = nw * GRP * CH
    ep = -(-e // quant) * quant
    pad = ep - e
    pad_idx = jnp.arange(pad, dtype=jnp.int32)
    src = jnp.concatenate([edge_index[0], pad_idx % n]).reshape(nw, ep // (nw * CH), CH)
    dst = jnp.concatenate([edge_index[1], n + pad_idx % (npad - n)]
                          ).reshape(nw, ep // (nw * CH), CH)

    aggregate = _make_aggregate(n, ep)
    layer = _make_layer(n)

    def layer_args(p):
        dm = p["W1"].shape[1]
        d2 = p["W2"].shape[1]
        return (
            jnp.reshape(1.0 + p["eps"], (1, 1)),
            _pad_to(p["W1"], (W, W)), _pad_to(jnp.reshape(p["b1"], (1, dm)), (1, W)),
            _pad_to(jnp.reshape(p["bn_mid_g"], (1, dm)), (1, W)),
            _pad_to(jnp.reshape(p["bn_mid_b"], (1, dm)), (1, W)),
            _pad_to(p["W2"], (W, W)),
            _pad_to(jnp.reshape(p["b2"], (1, d2)), (1, W)),
            _pad_to(jnp.reshape(p["bn_g"], (1, d2)), (1, W)),
            _pad_to(jnp.reshape(p["bn_b"], (1, d2)), (1, W)),
        )

    h = x
    for p in params["convs"][:-1]:
        acc = aggregate(h, src, dst, zero)
        h = layer(h, acc, *layer_args(p))

    d = params["lin1_W"].shape[0]
    do = params["lin2_W"].shape[1]
    acc = aggregate(h, src, dst, zero)
    return _make_layer_head(n, do)(
        h, acc, *layer_args(params["convs"][-1]),
        _pad_to(params["lin1_W"], (W, d)),
        jnp.reshape(params["lin1_b"], (1, d)),
        jnp.reshape(params["bn1_g"], (1, d)), jnp.reshape(params["bn1_b"], (1, d)),
        params["lin2_W"], jnp.reshape(params["lin2_b"], (1, do)),
    )


# speedup vs baseline: 11.8835x; 1.0131x over previous
import functools

import jax
import jax.numpy as jnp
from jax import lax
from jax.experimental import pallas as pl
from jax.experimental.pallas import tpu as pltpu
from jax.experimental.pallas import tpu_sc as plsc

NC = 2
NS = 16
CH = 64
GRP = 16
NB = 4
W = 128


@functools.cache
def _make_aggregate(n, e):
    nw = NC * NS
    epw = e // nw
    nch = epw // CH
    ng = nch // GRP
    assert e % (nw * GRP * CH) == 0 and ng >= 3
    npad = -(-n // (8 * NS)) * (8 * NS)
    rps = npad // NS

    mesh = plsc.VectorSubcoreMesh(core_axis_name="c", subcore_axis_name="s")

    def body(x_hbm, src_hbm, dst_hbm, zero_hbm, out_hbm,
             acc_sh, src_r, dst_r, b0, b1, b2, b3,
             gs0, gs1, gs2, gs3, ss0, ss1, ss2, ss3, ise, ide, ze):
        c = lax.axis_index("c")
        s = lax.axis_index("s")
        wid = s * NC + c
        bufs = [b0, b1, b2, b3]
        gs = [gs0, gs1, gs2, gs3]
        ss = [ss0, ss1, ss2, ss3]

        def i_start(g, r):
            off = pl.multiple_of(g * GRP, GRP)
            pltpu.async_copy(src_hbm.at[wid, pl.ds(off, GRP)], src_r.at[r], ise)
            pltpu.async_copy(dst_hbm.at[wid, pl.ds(off, GRP)], dst_r.at[r], ide)

        def i_wait(g, r):
            off = pl.multiple_of(g * GRP, GRP)
            pltpu.make_async_copy(src_hbm.at[wid, pl.ds(off, GRP)],
                                  src_r.at[r], ise).wait()
            pltpu.make_async_copy(dst_hbm.at[wid, pl.ds(off, GRP)],
                                  dst_r.at[r], ide).wait()

        def g_start(gr, j, b):
            pltpu.async_copy(x_hbm.at[src_r.at[gr, j]], bufs[b], gs[b])

        def slot(gr, j, b, swait=None, gst=None):
            bn = (b + 2) % NB
            if swait is not None:
                pltpu.make_async_copy(bufs[bn], acc_sh.at[dst_r.at[swait]],
                                      ss[bn]).wait()
            if gst is not None:
                g_start(gst[0], gst[1], bn)
            pltpu.make_async_copy(x_hbm.at[src_r.at[gr, j]], bufs[b],
                                  gs[b]).wait()
            pltpu.async_copy(bufs[b], acc_sh.at[dst_r.at[gr, j]], ss[b],
                             add=True)

        def group(g, gr, go, first=False, last=False):
            slot(gr, 0, 0,
                 swait=None if first else (go, GRP - 2), gst=(gr, 2))
            slot(gr, 1, 1,
                 swait=None if first else (go, GRP - 1), gst=(gr, 3))
            if not last:
                i_start(g + 1, go)
            for j in range(2, GRP - 2):
                slot(gr, j, j % NB, swait=(gr, j - 2), gst=(gr, j + 2))
            if not last:
                i_wait(g + 1, go)
            slot(gr, GRP - 2, (GRP - 2) % NB, swait=(gr, GRP - 4),
                 gst=None if last else (go, 0))
            slot(gr, GRP - 1, (GRP - 1) % NB, swait=(gr, GRP - 3),
                 gst=None if last else (go, 1))

        pltpu.async_copy(zero_hbm.at[pl.ds(s * rps, rps)],
                        acc_sh.at[pl.ds(s * rps, rps)], ze)
        i_start(0, 0)
        i_wait(0, 0)
        g_start(0, 0, 0)
        g_start(0, 1, 1)
        pltpu.make_async_copy(zero_hbm.at[pl.ds(s * rps, rps)],
                              acc_sh.at[pl.ds(s * rps, rps)], ze).wait()
        plsc.subcore_barrier()

        group(0, 0, 1, first=True)

        def mid(g, carry):
            gr = lax.rem(g, 2)
            group(g, gr, 1 - gr)
            return carry

        lax.fori_loop(1, ng - 1, mid, 0)

        gl = (ng - 1) % 2
        group(ng - 1, gl, 1 - gl, last=True)
        pltpu.make_async_copy(
            bufs[(GRP - 2) % NB],
            acc_sh.at[dst_r.at[gl, GRP - 2]],
            ss[(GRP - 2) % NB]).wait()
        pltpu.make_async_copy(
            bufs[(GRP - 1) % NB],
            acc_sh.at[dst_r.at[gl, GRP - 1]],
            ss[(GRP - 1) % NB]).wait()

        plsc.subcore_barrier()
        pltpu.sync_copy(acc_sh.at[pl.ds(s * rps, rps)],
                        out_hbm.at[c, pl.ds(s * rps, rps)])

    return pl.kernel(
        body,
        out_type=jax.ShapeDtypeStruct((NC, npad, W), jnp.float32),
        mesh=mesh,
        scratch_types=[
            pltpu.VMEM_SHARED((npad, W), jnp.float32),
            pltpu.VMEM((2, GRP, CH), jnp.int32),
            pltpu.VMEM((2, GRP, CH), jnp.int32),
        ] + [pltpu.VMEM((CH, W), jnp.float32)] * NB
          + [pltpu.SemaphoreType.DMA] * (2 * NB + 3),
    )


def _bn_in(t, g, b):
    mu = jnp.mean(t, axis=0, keepdims=True)
    var = jnp.mean((t - mu) * (t - mu), axis=0, keepdims=True)
    return (t - mu) / jnp.sqrt(var + 1e-5) * g + b


def _layer_math(n, h_ref, a_ref, eps_ref, w1_ref, b1_ref, gm_ref, bm_ref,
                w2_ref, b2_ref, g2_ref, bb2_ref):
    z = h_ref[...] * eps_ref[0, 0] + a_ref[0, :n] + a_ref[1, :n]
    t = jnp.dot(z, w1_ref[...], preferred_element_type=jnp.float32)
    t = _bn_in(t + b1_ref[...], gm_ref[...], bm_ref[...])
    t = jnp.maximum(t, 0.0)
    u = jnp.dot(t, w2_ref[...], preferred_element_type=jnp.float32)
    u = _bn_in(u + b2_ref[...], g2_ref[...], bb2_ref[...])
    return jnp.maximum(u, 0.0)


@functools.cache
def _make_layer(n):
    def body(h_ref, a_ref, eps_ref,
             w1_ref, b1_ref, gm_ref, bm_ref,
             w2_ref, b2_ref, g2_ref, bb2_ref, out_ref):
        out_ref[...] = _layer_math(n, h_ref, a_ref, eps_ref, w1_ref, b1_ref,
                                   gm_ref, bm_ref, w2_ref, b2_ref, g2_ref,
                                   bb2_ref)

    vmem = pl.BlockSpec(memory_space=pltpu.VMEM)
    smem = pl.BlockSpec(memory_space=pltpu.SMEM)
    return pl.pallas_call(
        body,
        out_shape=jax.ShapeDtypeStruct((n, W), jnp.float32),
        in_specs=[vmem, vmem, smem] + [vmem] * 8,
        out_specs=vmem,
    )


@functools.cache
def _make_layer_head(n, do):
    def body(h_ref, a_ref, eps_ref,
             w1_ref, b1_ref, gm_ref, bm_ref,
             w2_ref, b2_ref, g2_ref, bb2_ref,
             hw1_ref, hb1_ref, hg_ref, hb_ref, hw2_ref, hb2_ref, out_ref):
        h = _layer_math(n, h_ref, a_ref, eps_ref, w1_ref, b1_ref, gm_ref,
                        bm_ref, w2_ref, b2_ref, g2_ref, bb2_ref)
        t = jnp.dot(h, hw1_ref[...], preferred_element_type=jnp.float32)
        t = _bn_in(t + hb1_ref[...], hg_ref[...], hb_ref[...])
        t = jnp.maximum(t, 0.0)
        u = jnp.dot(t, hw2_ref[...], preferred_element_type=jnp.float32)
        u = u + hb2_ref[...]
        m = jnp.max(u, axis=-1, keepdims=True)
        ex = jnp.exp(u - m)
        lse = jnp.log(jnp.sum(ex, axis=-1, keepdims=True)) + m
        out_ref[...] = u - lse

    vmem = pl.BlockSpec(memory_space=pltpu.VMEM)
    smem = pl.BlockSpec(memory_space=pltpu.SMEM)
    return pl.pallas_call(
        body,
        out_shape=jax.ShapeDtypeStruct((n, do), jnp.float32),
        in_specs=[vmem, vmem, smem] + [vmem] * 14,
        out_specs=vmem,
    )


def _pad_to(a, shape):
    pads = [(0, t - s) for s, t in zip(a.shape, shape)]
    return jnp.pad(a, pads)


def kernel(x, edge_index, params):
    n, d_in = x.shape
    e = edge_index.shape[1]
    npad = -(-n // (8 * NS)) * (8 * NS)

    nw = NC * NS
    quant = nw * GRP * CH
    ep = -(-e // quant) * quant
    pad = ep - e
    pad_idx = jnp.arange(pad, dtype=jnp.int32)
    src = jnp.concatenate([edge_index[0], pad_idx % n]).reshape(nw, ep // (nw * CH), CH)
    dst = jnp.concatenate([edge_index[1], n + pad_idx % (npad - n)]
                          ).reshape(nw, ep // (nw * CH), CH)

    def layer_args(p):
        dm = p["W1"].shape[1]
        d2 = p["W2"].shape[1]
        return (
            jnp.reshape(1.0 + p["eps"], (1, 1)),
            _pad_to(p["W1"], (W, W)), _pad_to(jnp.reshape(p["b1"], (1, dm)), (1, W)),
            _pad_to(jnp.reshape(p["bn_mid_g"], (1, dm)), (1, W)),
            _pad_to(jnp.reshape(p["bn_mid_b"], (1, dm)), (1, W)),
            _pad_to(p["W2"], (W, W)),
            _pad_to(jnp.reshape(p["b2"], (1, d2)), (1, W)),
            _pad_to(jnp.reshape(p["bn_g"], (1, d2)), (1, W)),
            _pad_to(jnp.reshape(p["bn_b"], (1, d2)), (1, W)),
        )

    zero = jnp.zeros((npad, W), jnp.float32)
    aggregate = _make_aggregate(n, ep)
    layer = _make_layer(n)

    h = x
    for p in params["convs"][:-1]:
        acc = aggregate(h, src, dst, zero)
        h = layer(h, acc, *layer_args(p))

    dh = params["lin1_W"].shape[0]
    do = params["lin2_W"].shape[1]
    acc = aggregate(h, src, dst, zero)
    return _make_layer_head(n, do)(
        h, acc, *layer_args(params["convs"][-1]),
        _pad_to(params["lin1_W"], (W, dh)),
        jnp.reshape(params["lin1_b"], (1, dh)),
        jnp.reshape(params["bn1_g"], (1, dh)), jnp.reshape(params["bn1_b"], (1, dh)),
        params["lin2_W"], jnp.reshape(params["lin2_b"], (1, do)),
    )
